# baseline (speedup 1.0000x reference)
import functools

import jax
import jax.numpy as jnp
from jax import lax
from jax.experimental import pallas as pl
from jax.experimental.pallas import tpu as pltpu
from jax.experimental.pallas import tpu_sc as plsc

_N_CODES = 1000000
_CODE_CH = 32
_BATCH = 16384

_info = plsc.get_sparse_core_info()
_NC = _info.num_cores
_NS = _info.num_subcores
_NW = _NC * _NS
_B_PER_W = _BATCH // _NW
_CHUNK = 8
_NCHUNK = _B_PER_W // _CHUNK

_mesh = plsc.VectorSubcoreMesh(core_axis_name="c", subcore_axis_name="s")


@functools.partial(
    pl.kernel,
    mesh=_mesh,
    out_type=jax.ShapeDtypeStruct((_CODE_CH, _BATCH), jnp.float32),
    scratch_types=[
        pltpu.VMEM((_B_PER_W + 16,), jnp.int32),
        pltpu.VMEM((16,), jnp.int32),
        pltpu.VMEM((_CHUNK, 4, 8, 128), jnp.float32),
        pltpu.VMEM((_CHUNK, 4, 8, 128), jnp.float32),
        pltpu.VMEM((_CHUNK, 4, 8, 128), jnp.float32),
        pltpu.VMEM((_CODE_CH, _B_PER_W), jnp.float32),
        pltpu.SemaphoreType.DMA,
        pltpu.SemaphoreType.DMA,
        pltpu.SemaphoreType.DMA,
        pltpu.SemaphoreType.DMA,
    ],
    compiler_params=pltpu.CompilerParams(needs_layout_passes=False),
)
def _gather_kernel(idx_hbm, table_hbm, out_hbm, idx_v, slab_v,
                   buf_a, buf_b, buf_c, rows_v, sem_a, sem_b, sem_c, sem_o):
    wid = lax.axis_index("s") * _NC + lax.axis_index("c")
    base = wid * _B_PER_W

    pltpu.sync_copy(idx_hbm.at[pl.ds(base, _B_PER_W)],
                    idx_v.at[pl.ds(0, _B_PER_W)])

    lanes = lax.iota(jnp.int32, 16)
    slab_v[...] = lanes

    pats = []
    for h in range(2):
        c = lanes + h * 16
        pats.append((c >> 3, c & 7, c))

    def fire(ch, buf, sem):
        rv = idx_v[pl.ds(ch * _CHUNK, 16)]
        for k in range(_CHUNK):
            r = rv[k]
            mo = pl.multiple_of((r >> 7) << 7, 128)
            pltpu.async_copy(
                table_hbm.at[slab_v.at[pl.ds(0, 4)], :, pl.ds(mo, 128)],
                buf.at[k],
                sem,
            )

    def drain(sem):
        pltpu.make_async_copy(out_hbm.at[:, pl.ds(0, _B_PER_W)], rows_v,
                              sem).wait()
        pltpu.make_async_copy(out_hbm.at[:, pl.ds(0, _B_PER_W)], rows_v,
                              sem).wait()

    def extract(ch, buf):
        rv = idx_v[pl.ds(ch * _CHUNK, 16)]
        for k in range(_CHUNK):
            i = ch * _CHUNK + k
            r = rv[k]
            o = jnp.full((16,), r & 127, jnp.int32)
            ii = jnp.full((16,), i, jnp.int32)
            for g, s, c in pats:
                vals = plsc.load_gather(buf.at[k], [g, s, o])
                plsc.store_scatter(rows_v, [c, ii], vals)

    bufs = (buf_a, buf_b, buf_c)
    sems = (sem_a, sem_b, sem_c)

    fire(0, bufs[0], sems[0])
    fire(1, bufs[1], sems[1])

    def pbody(q, carry):
        for j in range(3):
            drain(sems[j])
            extract(3 * q + j, bufs[j])
            nxt = (j + 2) % 3
            fire(3 * q + j + 2, bufs[nxt], sems[nxt])
        return carry

    lax.fori_loop(0, (_NCHUNK - 4) // 3, pbody, 0)

    fire(_NCHUNK - 2, bufs[2], sems[2])
    drain(sems[0])
    extract(_NCHUNK - 4, bufs[0])
    fire(_NCHUNK - 1, bufs[0], sems[0])
    drain(sems[1])
    extract(_NCHUNK - 3, bufs[1])
    drain(sems[2])
    extract(_NCHUNK - 2, bufs[2])
    drain(sems[0])
    extract(_NCHUNK - 1, bufs[0])

    copies = [
        pltpu.async_copy(
            rows_v.at[pl.ds(8 * g, 8), :],
            out_hbm.at[pl.ds(8 * g, 8), pl.ds(base, _B_PER_W)],
            sem_o,
        )
        for g in range(_CODE_CH // 8)
    ]
    for cp in copies:
        cp.wait()


def kernel(idx, table):
    table3 = table.T.reshape(4, 8, _N_CODES)
    out_t = _gather_kernel(idx.reshape(_BATCH), table3)
    return out_t.T


# speedup vs baseline: 1.0207x; 1.0207x over previous
import functools

import jax
import jax.numpy as jnp
from jax import lax
from jax.experimental import pallas as pl
from jax.experimental.pallas import tpu as pltpu
from jax.experimental.pallas import tpu_sc as plsc

_N_CODES = 1000000
_CODE_CH = 32
_BATCH = 16384

_info = plsc.get_sparse_core_info()
_NC = _info.num_cores
_NS = _info.num_subcores
_NW = _NC * _NS
_B_PER_W = _BATCH // _NW
_CHUNK = 8
_NCHUNK = _B_PER_W // _CHUNK

_mesh = plsc.VectorSubcoreMesh(core_axis_name="c", subcore_axis_name="s")


@functools.partial(
    pl.kernel,
    mesh=_mesh,
    out_type=jax.ShapeDtypeStruct((_CODE_CH, _BATCH), jnp.float32),
    scratch_types=[
        pltpu.VMEM((_B_PER_W + 16,), jnp.int32),
        pltpu.VMEM((16,), jnp.int32),
        pltpu.VMEM((_CHUNK, 4, 8, 128), jnp.float32),
        pltpu.VMEM((_CHUNK, 4, 8, 128), jnp.float32),
        pltpu.VMEM((_CODE_CH, _B_PER_W), jnp.float32),
        pltpu.SemaphoreType.DMA,
        pltpu.SemaphoreType.DMA,
        pltpu.SemaphoreType.DMA,
    ],
    compiler_params=pltpu.CompilerParams(
        needs_layout_passes=False, skip_device_barrier=True
    ),
)
def _gather_kernel(idx_hbm, table_hbm, out_hbm, idx_v, slab_v,
                   buf_a, buf_b, rows_v, sem_a, sem_b, sem_o):
    wid = lax.axis_index("s") * _NC + lax.axis_index("c")
    base = wid * _B_PER_W

    pltpu.sync_copy(idx_hbm.at[pl.ds(base, _B_PER_W)],
                    idx_v.at[pl.ds(0, _B_PER_W)])

    lanes = lax.iota(jnp.int32, 16)
    slab_v[...] = lanes

    pats = []
    for h in range(2):
        c = lanes + h * 16
        pats.append((c >> 3, c & 7, c))

    def fire(ch, buf, sem):
        rv = idx_v[pl.ds(ch * _CHUNK, 16)]
        for k in range(_CHUNK):
            r = rv[k]
            mo = pl.multiple_of((r >> 7) << 7, 128)
            pltpu.async_copy(
                table_hbm.at[slab_v.at[pl.ds(0, 4)], :, pl.ds(mo, 128)],
                buf.at[k],
                sem,
            )

    def drain(sem):
        pltpu.make_async_copy(out_hbm.at[:, pl.ds(0, _B_PER_W)], rows_v,
                              sem).wait()
        pltpu.make_async_copy(out_hbm.at[:, pl.ds(0, _B_PER_W)], rows_v,
                              sem).wait()

    def extract(ch, buf):
        rv = idx_v[pl.ds(ch * _CHUNK, 16)]
        for k in range(_CHUNK):
            i = ch * _CHUNK + k
            r = rv[k]
            o = jnp.full((16,), r & 127, jnp.int32)
            ii = jnp.full((16,), i, jnp.int32)
            for g, s, c in pats:
                vals = plsc.load_gather(buf.at[k], [g, s, o])
                plsc.store_scatter(rows_v, [c, ii], vals)

    fire(0, buf_a, sem_a)

    def pbody(p, carry):
        fire(2 * p + 1, buf_b, sem_b)
        drain(sem_a)
        extract(2 * p, buf_a)
        fire(2 * p + 2, buf_a, sem_a)
        drain(sem_b)
        extract(2 * p + 1, buf_b)
        return carry

    lax.fori_loop(0, _NCHUNK // 2 - 1, pbody, 0)

    fire(_NCHUNK - 1, buf_b, sem_b)
    drain(sem_a)
    extract(_NCHUNK - 2, buf_a)
    drain(sem_b)
    extract(_NCHUNK - 1, buf_b)

    copies = [
        pltpu.async_copy(
            rows_v.at[pl.ds(8 * g, 8), :],
            out_hbm.at[pl.ds(8 * g, 8), pl.ds(base, _B_PER_W)],
            sem_o,
        )
        for g in range(_CODE_CH // 8)
    ]
    for cp in copies:
        cp.wait()


def kernel(idx, table):
    table3 = table.T.reshape(4, 8, _N_CODES)
    out_t = _gather_kernel(idx.reshape(_BATCH), table3)
    return out_t.T
